# Initial kernel scaffold; baseline (speedup 1.0000x reference)
import jax
import jax.numpy as jnp
from jax.experimental import pallas as pl


def kernel(instance, compsrc, maxinsnum):
    raise NotImplementedError("write your pallas kernel here")


# speedup vs baseline: 4.6372x; 4.6372x over previous
import functools

import jax
import jax.numpy as jnp
from jax import lax
from jax.experimental import pallas as pl
from jax.experimental.pallas import tpu as pltpu
from jax.experimental.pallas import tpu_sc as plsc

NUM_SEGMENTS = 200
COMP = 9
PADC = 16
NC = 2
NS = 16
CHUNK = 2048


def _sc_segment_sum(ids2, src16, zeros, hw):
    bz = ids2.shape[0]
    batches_per_core = bz // NC
    pix_per_tile = hw // NS
    n_chunks = pix_per_tile // CHUNK
    assert pix_per_tile % CHUNK == 0

    mesh = plsc.VectorSubcoreMesh(core_axis_name="c", subcore_axis_name="s")

    @functools.partial(
        pl.kernel,
        out_type=jax.ShapeDtypeStruct((bz, NUM_SEGMENTS, PADC), jnp.float32),
        mesh=mesh,
        compiler_params=pltpu.CompilerParams(use_tc_tiling_on_sc=False),
        scratch_types=[
            [pltpu.VMEM_SHARED((NUM_SEGMENTS, PADC), jnp.float32)
             for _ in range(batches_per_core)],
            pltpu.VMEM((CHUNK, PADC), jnp.float32),
            pltpu.VMEM((CHUNK,), jnp.int32),
        ],
    )
    def seg_sum(ids_hbm, src_hbm, zeros_hbm, out_hbm, accs, src_v, idx_v):
        c = lax.axis_index("c")
        s = lax.axis_index("s")

        @pl.when(s == 0)
        def _():
            for acc in accs:
                pltpu.sync_copy(zeros_hbm, acc)

        plsc.subcore_barrier()

        for local_b in range(batches_per_core):
            acc = accs[local_b]
            b = c * batches_per_core + local_b

            def chunk_body(i, _, acc=acc, b=b):
                p0 = s * pix_per_tile + i * CHUNK
                pltpu.sync_copy(src_hbm.at[b, pl.ds(p0, CHUNK)], src_v)
                pltpu.sync_copy(ids_hbm.at[b, pl.ds(p0, CHUNK)], idx_v)
                pltpu.sync_copy(src_v, acc.at[idx_v], add=True)
                return 0

            lax.fori_loop(0, n_chunks, chunk_body, 0)

        plsc.subcore_barrier()

        @pl.when(s == 0)
        def _():
            for local_b in range(batches_per_core):
                pltpu.sync_copy(accs[local_b],
                                out_hbm.at[c * batches_per_core + local_b])

    return seg_sum(ids2, src16, zeros)


def _tc_finish(x16):
    bz = x16.shape[0]

    def body(x_ref, o_ref):
        o_ref[...] = x_ref[:, :, :COMP] * jnp.float32(1.1)

    return pl.pallas_call(
        body,
        out_shape=jax.ShapeDtypeStruct((bz, NUM_SEGMENTS, COMP), jnp.float32),
    )(x16)


def kernel(instance, compsrc, maxinsnum):
    bz, _, h, w = instance.shape
    hw = h * w
    ids2 = instance.reshape(bz, hw)
    src16 = jnp.pad(compsrc.reshape(bz, hw, COMP),
                    ((0, 0), (0, 0), (0, PADC - COMP)))
    zeros = jnp.zeros((NUM_SEGMENTS, PADC), jnp.float32)
    sums16 = _sc_segment_sum(ids2, src16, zeros, hw)
    out = _tc_finish(sums16)
    return out.reshape(bz, NUM_SEGMENTS, 3, 3)
